# baseline (speedup 1.0000x reference)
import functools

import jax
import jax.numpy as jnp
from jax import lax
from jax.experimental import pallas as pl
from jax.experimental.pallas import tpu as pltpu
from jax.experimental.pallas import tpu_sc as plsc

EPS = 1e-5
NC = 2
NS = 16
NW = NC * NS
CHUNK = 128
DEGW = 128


def _make_deg(NP, CPT):
    RT = NP // NS
    mesh = plsc.VectorSubcoreMesh(core_axis_name="c", subcore_axis_name="s")

    @functools.partial(
        pl.kernel,
        out_type=jax.ShapeDtypeStruct((NC * NP, DEGW), jnp.float32),
        mesh=mesh,
        scratch_types=[
            pltpu.VMEM((CPT, CHUNK), jnp.int32),
            pltpu.VMEM((CHUNK, DEGW), jnp.float32),
            pltpu.VMEM_SHARED((NP, DEGW), jnp.float32),
        ],
    )
    def deg_kernel(dstp_hbm, zer_hbm, ones_hbm, out_hbm, dst_l, ones_l, acc):
        c = lax.axis_index("c")
        s = lax.axis_index("s")
        g = c * NS + s
        pltpu.sync_copy(zer_hbm, acc.at[pl.ds(s * RT, RT)])
        pltpu.sync_copy(ones_hbm, ones_l)
        pltpu.sync_copy(dstp_hbm.at[pl.ds(g * CPT, CPT)], dst_l)
        plsc.subcore_barrier()

        def body(j, carry):
            pltpu.sync_copy(ones_l, acc.at[dst_l.at[j]], add=True)
            return carry

        lax.fori_loop(0, CPT, body, 0)
        plsc.subcore_barrier()
        pltpu.sync_copy(acc.at[pl.ds(s * RT, RT)],
                        out_hbm.at[pl.ds(c * NP + s * RT, RT)])

    return deg_kernel


def _make_segsum(NP, D, t0, t1):
    RT = NP // NS
    HMAX = max(t0, t1) // 2
    mesh = plsc.VectorSubcoreMesh(core_axis_name="c", subcore_axis_name="s")

    @functools.partial(
        pl.kernel,
        out_type=jax.ShapeDtypeStruct((NC * NP, D), jnp.float32),
        mesh=mesh,
        scratch_types=[
            pltpu.VMEM((HMAX, CHUNK), jnp.int32),
            pltpu.VMEM((HMAX, CHUNK), jnp.int32),
            pltpu.VMEM((CHUNK, D), jnp.float32),
            pltpu.VMEM_SHARED((NP, D), jnp.float32),
            pltpu.SemaphoreType.DMA,
        ],
    )
    def segsum_kernel(y_hbm, srcp_hbm, dstp_hbm, zer_hbm, out_hbm,
                      src_l, dst_l, bufa, acc, sema):
        c = lax.axis_index("c")
        s = lax.axis_index("s")
        t_c = lax.select(c == 0, t0, t1)
        base = lax.select(c == 0, s * t0, NS * t0 + s * t1)
        HC = t_c // 2
        H = t_c // 4
        pltpu.sync_copy(zer_hbm, acc.at[pl.ds(s * RT, RT)])
        plsc.subcore_barrier()

        for half in range(2):
            off = pl.multiple_of(base + half * HC, 8)
            pltpu.sync_copy(srcp_hbm.at[pl.ds(off, HMAX)], src_l)
            pltpu.sync_copy(dstp_hbm.at[pl.ds(off, HMAX)], dst_l)

            def body(j, carry):
                pltpu.async_copy(y_hbm.at[src_l.at[j]], bufa, sema).wait()
                pltpu.sync_copy(bufa, acc.at[dst_l.at[j]], add=True)
                return carry

            lax.fori_loop(0, HC, body, 0)
        plsc.subcore_barrier()
        pltpu.sync_copy(acc.at[pl.ds(s * RT, RT)],
                        out_hbm.at[pl.ds(c * NP + s * RT, RT)])

    return segsum_kernel


def _tc_first(x, W1, dega, degb):
    N, D = x.shape

    def body(x_ref, w_ref, da_ref, db_ref, y_ref, dinv_ref):
        deg = da_ref[...] + db_ref[...] + 1.0
        dinv = lax.rsqrt(deg)
        dinv_ref[...] = dinv
        y_ref[...] = jnp.dot(x_ref[...], w_ref[...],
                             preferred_element_type=jnp.float32) * dinv

    return pl.pallas_call(
        body,
        out_shape=(jax.ShapeDtypeStruct((N, D), jnp.float32),
                   jax.ShapeDtypeStruct((N, 1), jnp.float32)),
    )(x, W1, dega, degb)


def _tc_mid(sa, sb, y, dinv, b, g, be, W):
    N, D = y.shape

    def body(sa_ref, sb_ref, y_ref, dinv_ref, b_ref, g_ref, be_ref, w_ref,
             out_ref):
        dinv = dinv_ref[...]
        conv = dinv * (sa_ref[...] + sb_ref[...] + y_ref[...]) + b_ref[...]
        mu = jnp.mean(conv, axis=0, keepdims=True)
        var = jnp.mean((conv - mu) ** 2, axis=0, keepdims=True)
        h = (conv - mu) * lax.rsqrt(var + EPS) * g_ref[...] + be_ref[...]
        h = jnp.maximum(h, 0.0)
        out_ref[...] = jnp.dot(h, w_ref[...],
                               preferred_element_type=jnp.float32) * dinv

    return pl.pallas_call(
        body,
        out_shape=jax.ShapeDtypeStruct((N, D), jnp.float32),
    )(sa, sb, y, dinv, b, g, be, W)


def _tc_last(sa, sb, y, dinv, b):
    N, D = y.shape

    def body(sa_ref, sb_ref, y_ref, dinv_ref, b_ref, out_ref):
        o = dinv_ref[...] * (sa_ref[...] + sb_ref[...] + y_ref[...]) + b_ref[...]
        m = jnp.max(o, axis=1, keepdims=True)
        lse = jnp.log(jnp.sum(jnp.exp(o - m), axis=1, keepdims=True)) + m
        out_ref[...] = o - lse

    return pl.pallas_call(
        body,
        out_shape=jax.ShapeDtypeStruct((N, D), jnp.float32),
    )(sa, sb, y, dinv, b)


def kernel(x, edge_index, W1, b1, g1, be1, W2, b2, g2, be2, W3, b3):
    N, D = x.shape
    E = edge_index.shape[1]
    TP = 16 * (-(-E // (CHUNK * NS * 16)))
    t0 = max(16, min(TP - 16, 16 * round(0.3 * TP / 16)))
    t1 = TP - t0
    HMAX = max(t0, t1) // 2
    CPTD = TP // 2
    EP = NS * TP * CHUNK
    TOTP = NS * TP + HMAX
    NP = ((N + 1 + NS * 8 - 1) // (NS * 8)) * (NS * 8)
    RT = NP // NS

    src = edge_index[0].astype(jnp.int32)
    dst = edge_index[1].astype(jnp.int32)
    pad = EP - E
    tail = HMAX * CHUNK
    srcp = jnp.concatenate(
        [src, jnp.zeros((pad + tail,), jnp.int32)]).reshape(TOTP, CHUNK)
    dstp = jnp.concatenate(
        [dst, jnp.full((pad,), N, jnp.int32),
         jnp.zeros((tail,), jnp.int32)]).reshape(TOTP, CHUNK)

    zeros_acc = jnp.zeros((RT, D), jnp.float32)
    zeros_deg = zeros_acc if DEGW == D else jnp.zeros((RT, DEGW), jnp.float32)
    ones_deg = jnp.ones((CHUNK, DEGW), jnp.float32)

    deg_kernel = _make_deg(NP, CPTD)
    segsum = _make_segsum(NP, D, t0, t1)

    degf = deg_kernel(dstp, zeros_deg, ones_deg)
    dega = degf[0:N, 0:1]
    degb = degf[NP:NP + N, 0:1]

    y1, dinv = _tc_first(x, W1, dega, degb)
    s1 = segsum(y1, srcp, dstp, zeros_acc)
    y2 = _tc_mid(s1[0:N], s1[NP:NP + N], y1, dinv,
                 b1.reshape(1, D), g1.reshape(1, D), be1.reshape(1, D), W2)
    s2 = segsum(y2, srcp, dstp, zeros_acc)
    y3 = _tc_mid(s2[0:N], s2[NP:NP + N], y2, dinv,
                 b2.reshape(1, D), g2.reshape(1, D), be2.reshape(1, D), W3)
    s3 = segsum(y3, srcp, dstp, zeros_acc)
    out = _tc_last(s3[0:N], s3[NP:NP + N], y3, dinv, b3.reshape(1, D))
    return out


# speedup vs baseline: 1.9231x; 1.9231x over previous
import functools

import jax
import jax.numpy as jnp
from jax import lax
from jax.experimental import pallas as pl
from jax.experimental.pallas import tpu as pltpu
from jax.experimental.pallas import tpu_sc as plsc

EPS = 1e-5
NC = 2
NS = 16
NW = NC * NS
CHUNK = 128
DEGW = 128


def _make_deg(NP, CPT):
    RT = NP // NS
    mesh = plsc.VectorSubcoreMesh(core_axis_name="c", subcore_axis_name="s")

    @functools.partial(
        pl.kernel,
        out_type=jax.ShapeDtypeStruct((NC * NP, DEGW), jnp.float32),
        mesh=mesh,
        scratch_types=[
            pltpu.VMEM((CPT, CHUNK), jnp.int32),
            pltpu.VMEM((CHUNK, DEGW), jnp.float32),
            pltpu.VMEM_SHARED((NP, DEGW), jnp.float32),
        ],
    )
    def deg_kernel(dstp_hbm, zer_hbm, ones_hbm, out_hbm, dst_l, ones_l, acc):
        c = lax.axis_index("c")
        s = lax.axis_index("s")
        g = c * NS + s
        pltpu.sync_copy(zer_hbm, acc.at[pl.ds(s * RT, RT)])
        pltpu.sync_copy(ones_hbm, ones_l)
        pltpu.sync_copy(dstp_hbm.at[g], dst_l)
        plsc.subcore_barrier()

        def body(j, carry):
            pltpu.sync_copy(ones_l, acc.at[dst_l.at[j]], add=True)
            return carry

        lax.fori_loop(0, CPT, body, 0)
        plsc.subcore_barrier()
        pltpu.sync_copy(acc.at[pl.ds(s * RT, RT)],
                        out_hbm.at[pl.ds(c * NP + s * RT, RT)])

    return deg_kernel


def _make_segsum(NP, D, t0, t1):
    RT = NP // NS
    TM = max(t0, t1)
    mesh = plsc.VectorSubcoreMesh(core_axis_name="c", subcore_axis_name="s")

    @functools.partial(
        pl.kernel,
        out_type=jax.ShapeDtypeStruct((NC * NP, D), jnp.float32),
        mesh=mesh,
        scratch_types=[
            pltpu.VMEM((TM, CHUNK), jnp.int32),
            pltpu.VMEM((TM, CHUNK), jnp.int32),
            pltpu.VMEM((CHUNK, D), jnp.float32),
            pltpu.VMEM_SHARED((NP, D), jnp.float32),
            pltpu.SemaphoreType.DMA,
        ],
    )
    def segsum_kernel(y_hbm, srcp0_hbm, dstp0_hbm, srcp1_hbm, dstp1_hbm,
                      zer_hbm, out_hbm, src_l, dst_l, bufa, acc, sema):
        c = lax.axis_index("c")
        s = lax.axis_index("s")
        pltpu.sync_copy(zer_hbm, acc.at[pl.ds(s * RT, RT)])
        plsc.subcore_barrier()

        def body(j, carry):
            pltpu.async_copy(y_hbm.at[src_l.at[j]], bufa, sema).wait()
            pltpu.sync_copy(bufa, acc.at[dst_l.at[j]], add=True)
            return carry

        @pl.when(c == 0)
        def _():
            pltpu.sync_copy(srcp0_hbm.at[s], src_l.at[pl.ds(0, t0)])
            pltpu.sync_copy(dstp0_hbm.at[s], dst_l.at[pl.ds(0, t0)])
            lax.fori_loop(0, t0, body, 0)

        @pl.when(c == 1)
        def _():
            pltpu.sync_copy(srcp1_hbm.at[s], src_l.at[pl.ds(0, t1)])
            pltpu.sync_copy(dstp1_hbm.at[s], dst_l.at[pl.ds(0, t1)])
            lax.fori_loop(0, t1, body, 0)

        plsc.subcore_barrier()
        pltpu.sync_copy(acc.at[pl.ds(s * RT, RT)],
                        out_hbm.at[pl.ds(c * NP + s * RT, RT)])

    return segsum_kernel


def _tc_first(x, W1, dega, degb):
    N, D = x.shape

    def body(x_ref, w_ref, da_ref, db_ref, y_ref, dinv_ref):
        deg = da_ref[...] + db_ref[...] + 1.0
        dinv = lax.rsqrt(deg)
        dinv_ref[...] = dinv
        y_ref[...] = jnp.dot(x_ref[...], w_ref[...],
                             preferred_element_type=jnp.float32) * dinv

    return pl.pallas_call(
        body,
        out_shape=(jax.ShapeDtypeStruct((N, D), jnp.float32),
                   jax.ShapeDtypeStruct((N, 1), jnp.float32)),
    )(x, W1, dega, degb)


def _tc_mid(sa, sb, y, dinv, b, g, be, W):
    N, D = y.shape

    def body(sa_ref, sb_ref, y_ref, dinv_ref, b_ref, g_ref, be_ref, w_ref,
             out_ref):
        dinv = dinv_ref[...]
        conv = dinv * (sa_ref[...] + sb_ref[...] + y_ref[...]) + b_ref[...]
        mu = jnp.mean(conv, axis=0, keepdims=True)
        var = jnp.mean((conv - mu) ** 2, axis=0, keepdims=True)
        h = (conv - mu) * lax.rsqrt(var + EPS) * g_ref[...] + be_ref[...]
        h = jnp.maximum(h, 0.0)
        out_ref[...] = jnp.dot(h, w_ref[...],
                               preferred_element_type=jnp.float32) * dinv

    return pl.pallas_call(
        body,
        out_shape=jax.ShapeDtypeStruct((N, D), jnp.float32),
    )(sa, sb, y, dinv, b, g, be, W)


def _tc_last(sa, sb, y, dinv, b):
    N, D = y.shape

    def body(sa_ref, sb_ref, y_ref, dinv_ref, b_ref, out_ref):
        o = dinv_ref[...] * (sa_ref[...] + sb_ref[...] + y_ref[...]) + b_ref[...]
        m = jnp.max(o, axis=1, keepdims=True)
        lse = jnp.log(jnp.sum(jnp.exp(o - m), axis=1, keepdims=True)) + m
        out_ref[...] = o - lse

    return pl.pallas_call(
        body,
        out_shape=jax.ShapeDtypeStruct((N, D), jnp.float32),
    )(sa, sb, y, dinv, b)


def kernel(x, edge_index, W1, b1, g1, be1, W2, b2, g2, be2, W3, b3):
    N, D = x.shape
    E = edge_index.shape[1]
    TP = 2 * (-(-E // (CHUNK * NS * 2)))
    t0 = max(1, min(TP - 1, round(0.7 * TP)))
    t1 = TP - t0
    CPTD = TP // 2
    EP = NS * TP * CHUNK
    NP = ((N + 1 + NS * 8 - 1) // (NS * 8)) * (NS * 8)
    RT = NP // NS

    src = edge_index[0].astype(jnp.int32)
    dst = edge_index[1].astype(jnp.int32)
    pad = EP - E
    srcp = jnp.concatenate([src, jnp.zeros((pad,), jnp.int32)])
    dstp = jnp.concatenate([dst, jnp.full((pad,), N, jnp.int32)])
    cut = NS * t0 * CHUNK
    srcp0 = srcp[:cut].reshape(NS, t0, CHUNK)
    srcp1 = srcp[cut:].reshape(NS, t1, CHUNK)
    dstp0 = dstp[:cut].reshape(NS, t0, CHUNK)
    dstp1 = dstp[cut:].reshape(NS, t1, CHUNK)
    dstflat = dstp.reshape(NW, CPTD, CHUNK)

    zeros_acc = jnp.zeros((RT, D), jnp.float32)
    zeros_deg = zeros_acc if DEGW == D else jnp.zeros((RT, DEGW), jnp.float32)
    ones_deg = jnp.ones((CHUNK, DEGW), jnp.float32)

    deg_kernel = _make_deg(NP, CPTD)
    segsum = _make_segsum(NP, D, t0, t1)

    degf = deg_kernel(dstflat, zeros_deg, ones_deg)
    dega = degf[0:N, 0:1]
    degb = degf[NP:NP + N, 0:1]

    y1, dinv = _tc_first(x, W1, dega, degb)
    s1 = segsum(y1, srcp0, dstp0, srcp1, dstp1, zeros_acc)
    y2 = _tc_mid(s1[0:N], s1[NP:NP + N], y1, dinv,
                 b1.reshape(1, D), g1.reshape(1, D), be1.reshape(1, D), W2)
    s2 = segsum(y2, srcp0, dstp0, srcp1, dstp1, zeros_acc)
    y3 = _tc_mid(s2[0:N], s2[NP:NP + N], y2, dinv,
                 b2.reshape(1, D), g2.reshape(1, D), be2.reshape(1, D), W3)
    s3 = segsum(y3, srcp0, dstp0, srcp1, dstp1, zeros_acc)
    out = _tc_last(s3[0:N], s3[NP:NP + N], y3, dinv, b3.reshape(1, D))
    return out


# speedup vs baseline: 1.9714x; 1.0251x over previous
import functools

import jax
import jax.numpy as jnp
from jax import lax
from jax.experimental import pallas as pl
from jax.experimental.pallas import tpu as pltpu
from jax.experimental.pallas import tpu_sc as plsc

EPS = 1e-5
NC = 2
NS = 16
NW = NC * NS
CHUNK = 128
DEGW = 128


def _make_deg(NP, CPT):
    RT = NP // NS
    mesh = plsc.VectorSubcoreMesh(core_axis_name="c", subcore_axis_name="s")

    @functools.partial(
        pl.kernel,
        out_type=jax.ShapeDtypeStruct((NC * NP, DEGW), jnp.float32),
        mesh=mesh,
        scratch_types=[
            pltpu.VMEM((CPT, CHUNK), jnp.int32),
            pltpu.VMEM((CHUNK, DEGW), jnp.float32),
            pltpu.VMEM_SHARED((NP, DEGW), jnp.float32),
        ],
    )
    def deg_kernel(dstp_hbm, zer_hbm, ones_hbm, out_hbm, dst_l, ones_l, acc):
        c = lax.axis_index("c")
        s = lax.axis_index("s")
        g = c * NS + s
        pltpu.sync_copy(zer_hbm, acc.at[pl.ds(s * RT, RT)])
        pltpu.sync_copy(ones_hbm, ones_l)
        pltpu.sync_copy(dstp_hbm.at[g], dst_l)
        plsc.subcore_barrier()

        def body(j, carry):
            pltpu.sync_copy(ones_l, acc.at[dst_l.at[j]], add=True)
            return carry

        lax.fori_loop(0, CPT, body, 0)
        plsc.subcore_barrier()
        pltpu.sync_copy(acc.at[pl.ds(s * RT, RT)],
                        out_hbm.at[pl.ds(c * NP + s * RT, RT)])

    return deg_kernel


def _make_segsum(NP, D, t0, t1):
    RT = NP // NS
    HC = t0 // 2
    H = HC // 2
    TM = max(HC, t1)
    mesh = plsc.VectorSubcoreMesh(core_axis_name="c", subcore_axis_name="s")

    @functools.partial(
        pl.kernel,
        out_type=jax.ShapeDtypeStruct((NC * NP, D), jnp.float32),
        mesh=mesh,
        scratch_types=[
            pltpu.VMEM((TM, CHUNK), jnp.int32),
            pltpu.VMEM((TM, CHUNK), jnp.int32),
            pltpu.VMEM((CHUNK, D), jnp.float32),
            pltpu.VMEM((CHUNK, D), jnp.float32),
            pltpu.VMEM_SHARED((NP, D), jnp.float32),
            pltpu.SemaphoreType.DMA,
            pltpu.SemaphoreType.DMA,
        ],
    )
    def segsum_kernel(y_hbm, srcp0_hbm, dstp0_hbm, srcp1_hbm, dstp1_hbm,
                      zer_hbm, out_hbm, src_l, dst_l, bufa, bufb, acc,
                      sema, semb):
        c = lax.axis_index("c")
        s = lax.axis_index("s")
        pltpu.sync_copy(zer_hbm, acc.at[pl.ds(s * RT, RT)])
        plsc.subcore_barrier()

        @pl.when(c == 0)
        def _():
            for half in range(2):
                pltpu.sync_copy(srcp0_hbm.at[s, pl.ds(half * HC, HC)],
                                src_l.at[pl.ds(0, HC)])
                pltpu.sync_copy(dstp0_hbm.at[s, pl.ds(half * HC, HC)],
                                dst_l.at[pl.ds(0, HC)])
                pltpu.async_copy(y_hbm.at[src_l.at[0]], bufa, sema)

                def body(it, carry):
                    j = it * 2
                    pltpu.make_async_copy(y_hbm.at[src_l.at[j]], bufa,
                                          sema).wait()
                    pltpu.async_copy(y_hbm.at[src_l.at[j + 1]], bufb, semb)
                    pltpu.sync_copy(bufa, acc.at[dst_l.at[j]], add=True)
                    pltpu.make_async_copy(y_hbm.at[src_l.at[j]], bufb,
                                          semb).wait()

                    @pl.when(it + 1 < H)
                    def _():
                        pltpu.async_copy(y_hbm.at[src_l.at[j + 2]], bufa,
                                         sema)

                    pltpu.sync_copy(bufb, acc.at[dst_l.at[j + 1]], add=True)
                    return carry

                lax.fori_loop(0, H, body, 0)

        @pl.when(c == 1)
        def _():
            pltpu.sync_copy(srcp1_hbm.at[s], src_l.at[pl.ds(0, t1)])
            pltpu.sync_copy(dstp1_hbm.at[s], dst_l.at[pl.ds(0, t1)])

            def body(j, carry):
                pltpu.async_copy(y_hbm.at[src_l.at[j]], bufa, sema).wait()
                pltpu.sync_copy(bufa, acc.at[dst_l.at[j]], add=True)
                return carry

            lax.fori_loop(0, t1, body, 0)

        plsc.subcore_barrier()
        pltpu.sync_copy(acc.at[pl.ds(s * RT, RT)],
                        out_hbm.at[pl.ds(c * NP + s * RT, RT)])

    return segsum_kernel


def _tc_first(x, W1, dega, degb):
    N, D = x.shape

    def body(x_ref, w_ref, da_ref, db_ref, y_ref, dinv_ref):
        deg = da_ref[...] + db_ref[...] + 1.0
        dinv = lax.rsqrt(deg)
        dinv_ref[...] = dinv
        y_ref[...] = jnp.dot(x_ref[...], w_ref[...],
                             preferred_element_type=jnp.float32) * dinv

    return pl.pallas_call(
        body,
        out_shape=(jax.ShapeDtypeStruct((N, D), jnp.float32),
                   jax.ShapeDtypeStruct((N, 1), jnp.float32)),
    )(x, W1, dega, degb)


def _tc_mid(sa, sb, y, dinv, b, g, be, W):
    N, D = y.shape

    def body(sa_ref, sb_ref, y_ref, dinv_ref, b_ref, g_ref, be_ref, w_ref,
             out_ref):
        dinv = dinv_ref[...]
        conv = dinv * (sa_ref[...] + sb_ref[...] + y_ref[...]) + b_ref[...]
        mu = jnp.mean(conv, axis=0, keepdims=True)
        var = jnp.mean((conv - mu) ** 2, axis=0, keepdims=True)
        h = (conv - mu) * lax.rsqrt(var + EPS) * g_ref[...] + be_ref[...]
        h = jnp.maximum(h, 0.0)
        out_ref[...] = jnp.dot(h, w_ref[...],
                               preferred_element_type=jnp.float32) * dinv

    return pl.pallas_call(
        body,
        out_shape=jax.ShapeDtypeStruct((N, D), jnp.float32),
    )(sa, sb, y, dinv, b, g, be, W)


def _tc_last(sa, sb, y, dinv, b):
    N, D = y.shape

    def body(sa_ref, sb_ref, y_ref, dinv_ref, b_ref, out_ref):
        o = dinv_ref[...] * (sa_ref[...] + sb_ref[...] + y_ref[...]) + b_ref[...]
        m = jnp.max(o, axis=1, keepdims=True)
        lse = jnp.log(jnp.sum(jnp.exp(o - m), axis=1, keepdims=True)) + m
        out_ref[...] = o - lse

    return pl.pallas_call(
        body,
        out_shape=jax.ShapeDtypeStruct((N, D), jnp.float32),
    )(sa, sb, y, dinv, b)


def kernel(x, edge_index, W1, b1, g1, be1, W2, b2, g2, be2, W3, b3):
    N, D = x.shape
    E = edge_index.shape[1]
    TP = 2 * (-(-E // (CHUNK * NS * 2)))
    t0 = max(4, min(TP - 1, 4 * round(0.71 * TP / 4)))
    t1 = TP - t0
    CPTD = TP // 2
    EP = NS * TP * CHUNK
    NP = ((N + 1 + NS * 8 - 1) // (NS * 8)) * (NS * 8)
    RT = NP // NS

    src = edge_index[0].astype(jnp.int32)
    dst = edge_index[1].astype(jnp.int32)
    pad = EP - E
    srcp = jnp.concatenate([src, jnp.zeros((pad,), jnp.int32)])
    dstp = jnp.concatenate([dst, jnp.full((pad,), N, jnp.int32)])
    cut = NS * t0 * CHUNK
    srcp0 = srcp[:cut].reshape(NS, t0, CHUNK)
    srcp1 = srcp[cut:].reshape(NS, t1, CHUNK)
    dstp0 = dstp[:cut].reshape(NS, t0, CHUNK)
    dstp1 = dstp[cut:].reshape(NS, t1, CHUNK)
    dstflat = dstp.reshape(NW, CPTD, CHUNK)

    zeros_acc = jnp.zeros((RT, D), jnp.float32)
    zeros_deg = zeros_acc if DEGW == D else jnp.zeros((RT, DEGW), jnp.float32)
    ones_deg = jnp.ones((CHUNK, DEGW), jnp.float32)

    deg_kernel = _make_deg(NP, CPTD)
    segsum = _make_segsum(NP, D, t0, t1)

    degf = deg_kernel(dstflat, zeros_deg, ones_deg)
    dega = degf[0:N, 0:1]
    degb = degf[NP:NP + N, 0:1]

    y1, dinv = _tc_first(x, W1, dega, degb)
    s1 = segsum(y1, srcp0, dstp0, srcp1, dstp1, zeros_acc)
    y2 = _tc_mid(s1[0:N], s1[NP:NP + N], y1, dinv,
                 b1.reshape(1, D), g1.reshape(1, D), be1.reshape(1, D), W2)
    s2 = segsum(y2, srcp0, dstp0, srcp1, dstp1, zeros_acc)
    y3 = _tc_mid(s2[0:N], s2[NP:NP + N], y2, dinv,
                 b2.reshape(1, D), g2.reshape(1, D), be2.reshape(1, D), W3)
    s3 = segsum(y3, srcp0, dstp0, srcp1, dstp1, zeros_acc)
    out = _tc_last(s3[0:N], s3[NP:NP + N], y3, dinv, b3.reshape(1, D))
    return out


# speedup vs baseline: 2.0419x; 1.0358x over previous
import functools

import jax
import jax.numpy as jnp
from jax import lax
from jax.experimental import pallas as pl
from jax.experimental.pallas import tpu as pltpu
from jax.experimental.pallas import tpu_sc as plsc

EPS = 1e-5
NC = 2
NS = 16
NW = NC * NS
CHUNK = 128
DEGW = 128


def _make_deg(NP, CPT):
    RT = NP // NS
    mesh = plsc.VectorSubcoreMesh(core_axis_name="c", subcore_axis_name="s")

    @functools.partial(
        pl.kernel,
        out_type=jax.ShapeDtypeStruct((NC * NP, DEGW), jnp.float32),
        mesh=mesh,
        scratch_types=[
            pltpu.VMEM((CPT, CHUNK), jnp.int32),
            pltpu.VMEM((CHUNK, DEGW), jnp.float32),
            pltpu.VMEM_SHARED((NP, DEGW), jnp.float32),
        ],
    )
    def deg_kernel(dstp_hbm, zer_hbm, ones_hbm, out_hbm, dst_l, ones_l, acc):
        c = lax.axis_index("c")
        s = lax.axis_index("s")
        g = c * NS + s
        pltpu.sync_copy(zer_hbm, acc.at[pl.ds(s * RT, RT)])
        pltpu.sync_copy(ones_hbm, ones_l)
        pltpu.sync_copy(dstp_hbm.at[g], dst_l)
        plsc.subcore_barrier()

        def body(j, carry):
            pltpu.sync_copy(ones_l, acc.at[dst_l.at[j]], add=True)
            return carry

        lax.fori_loop(0, CPT, body, 0)
        plsc.subcore_barrier()
        pltpu.sync_copy(acc.at[pl.ds(s * RT, RT)],
                        out_hbm.at[pl.ds(c * NP + s * RT, RT)])

    return deg_kernel


def _make_segsum(NP, D, t0, t1):
    RT = NP // NS
    HC = t0 // 2
    H = HC // 2
    TM = max(HC, t1)
    mesh = plsc.VectorSubcoreMesh(core_axis_name="c", subcore_axis_name="s")

    @functools.partial(
        pl.kernel,
        out_type=jax.ShapeDtypeStruct((NC * NP, D), jnp.float32),
        mesh=mesh,
        scratch_types=[
            pltpu.VMEM((TM, CHUNK), jnp.int32),
            pltpu.VMEM((TM, CHUNK), jnp.int32),
            pltpu.VMEM((CHUNK, D), jnp.float32),
            pltpu.VMEM((CHUNK, D), jnp.float32),
            pltpu.VMEM_SHARED((NP, D), jnp.float32),
            pltpu.SemaphoreType.DMA,
            pltpu.SemaphoreType.DMA,
        ],
    )
    def segsum_kernel(y_hbm, srcp0_hbm, dstp0_hbm, srcp1_hbm, dstp1_hbm,
                      zer_hbm, out_hbm, src_l, dst_l, bufa, bufb, acc,
                      sema, semb):
        c = lax.axis_index("c")
        s = lax.axis_index("s")
        pltpu.sync_copy(zer_hbm, acc.at[pl.ds(s * RT, RT)])
        plsc.subcore_barrier()

        @pl.when(c == 0)
        def _():
            for half in range(2):
                pltpu.sync_copy(srcp0_hbm.at[s, pl.ds(half * HC, HC)],
                                src_l.at[pl.ds(0, HC)])
                pltpu.sync_copy(dstp0_hbm.at[s, pl.ds(half * HC, HC)],
                                dst_l.at[pl.ds(0, HC)])
                pltpu.async_copy(y_hbm.at[src_l.at[0]], bufa, sema)

                def body(it, carry):
                    j = it * 2
                    pltpu.make_async_copy(y_hbm.at[src_l.at[j]], bufa,
                                          sema).wait()
                    pltpu.async_copy(y_hbm.at[src_l.at[j + 1]], bufb, semb)
                    pltpu.sync_copy(bufa, acc.at[dst_l.at[j]], add=True)
                    pltpu.make_async_copy(y_hbm.at[src_l.at[j]], bufb,
                                          semb).wait()

                    @pl.when(it + 1 < H)
                    def _():
                        pltpu.async_copy(y_hbm.at[src_l.at[j + 2]], bufa,
                                         sema)

                    pltpu.sync_copy(bufb, acc.at[dst_l.at[j + 1]], add=True)
                    return carry

                lax.fori_loop(0, H, body, 0)

        @pl.when(c == 1)
        def _():
            pltpu.sync_copy(srcp1_hbm.at[s], src_l.at[pl.ds(0, t1)])
            pltpu.sync_copy(dstp1_hbm.at[s], dst_l.at[pl.ds(0, t1)])

            def body(j, carry):
                pltpu.async_copy(y_hbm.at[src_l.at[j]], bufa, sema).wait()
                pltpu.sync_copy(bufa, acc.at[dst_l.at[j]], add=True)
                return carry

            lax.fori_loop(0, t1, body, 0)

        plsc.subcore_barrier()
        pltpu.sync_copy(acc.at[pl.ds(s * RT, RT)],
                        out_hbm.at[pl.ds(c * NP + s * RT, RT)])

    return segsum_kernel


def _tc_first(x, W1, dega, degb):
    N, D = x.shape

    def body(x_ref, w_ref, da_ref, db_ref, y_ref, dinv_ref):
        deg = da_ref[...] + db_ref[...] + 1.0
        dinv = lax.rsqrt(deg)
        dinv_ref[...] = dinv
        y_ref[...] = jnp.dot(x_ref[...], w_ref[...],
                             preferred_element_type=jnp.float32) * dinv

    return pl.pallas_call(
        body,
        out_shape=(jax.ShapeDtypeStruct((N, D), jnp.float32),
                   jax.ShapeDtypeStruct((N, 1), jnp.float32)),
    )(x, W1, dega, degb)


def _tc_mid(sa, sb, y, dinv, b, g, be, W):
    N, D = y.shape

    def body(sa_ref, sb_ref, y_ref, dinv_ref, b_ref, g_ref, be_ref, w_ref,
             out_ref):
        dinv = dinv_ref[...]
        conv = dinv * (sa_ref[...] + sb_ref[...] + y_ref[...]) + b_ref[...]
        mu = jnp.mean(conv, axis=0, keepdims=True)
        var = jnp.mean((conv - mu) ** 2, axis=0, keepdims=True)
        h = (conv - mu) * lax.rsqrt(var + EPS) * g_ref[...] + be_ref[...]
        h = jnp.maximum(h, 0.0)
        out_ref[...] = jnp.dot(h, w_ref[...],
                               preferred_element_type=jnp.float32) * dinv

    return pl.pallas_call(
        body,
        out_shape=jax.ShapeDtypeStruct((N, D), jnp.float32),
    )(sa, sb, y, dinv, b, g, be, W)


def _tc_last(sa, sb, y, dinv, b):
    N, D = y.shape

    def body(sa_ref, sb_ref, y_ref, dinv_ref, b_ref, out_ref):
        o = dinv_ref[...] * (sa_ref[...] + sb_ref[...] + y_ref[...]) + b_ref[...]
        m = jnp.max(o, axis=1, keepdims=True)
        lse = jnp.log(jnp.sum(jnp.exp(o - m), axis=1, keepdims=True)) + m
        out_ref[...] = o - lse

    return pl.pallas_call(
        body,
        out_shape=jax.ShapeDtypeStruct((N, D), jnp.float32),
    )(sa, sb, y, dinv, b)


def kernel(x, edge_index, W1, b1, g1, be1, W2, b2, g2, be2, W3, b3):
    N, D = x.shape
    E = edge_index.shape[1]
    TP = 2 * (-(-E // (CHUNK * NS * 2)))
    t0 = max(16, min(TP - 1, 16 * round(0.785 * TP / 16)))
    t1 = TP - t0
    CPTD = TP // 2
    EP = NS * TP * CHUNK
    NP = ((N + 1 + NS * 8 - 1) // (NS * 8)) * (NS * 8)
    RT = NP // NS

    src = edge_index[0].astype(jnp.int32)
    dst = edge_index[1].astype(jnp.int32)
    pad = EP - E
    srcp = jnp.concatenate([src, jnp.zeros((pad,), jnp.int32)])
    dstp = jnp.concatenate([dst, jnp.full((pad,), N, jnp.int32)])
    cut = NS * t0 * CHUNK
    srcp0 = srcp[:cut].reshape(NS, t0, CHUNK)
    srcp1 = srcp[cut:].reshape(NS, t1, CHUNK)
    dstp0 = dstp[:cut].reshape(NS, t0, CHUNK)
    dstp1 = dstp[cut:].reshape(NS, t1, CHUNK)
    dstflat = dstp.reshape(NW, CPTD, CHUNK)

    zeros_acc = jnp.zeros((RT, D), jnp.float32)
    zeros_deg = zeros_acc if DEGW == D else jnp.zeros((RT, DEGW), jnp.float32)
    ones_deg = jnp.ones((CHUNK, DEGW), jnp.float32)

    deg_kernel = _make_deg(NP, CPTD)
    segsum = _make_segsum(NP, D, t0, t1)

    degf = deg_kernel(dstflat, zeros_deg, ones_deg)
    dega = degf[0:N, 0:1]
    degb = degf[NP:NP + N, 0:1]

    y1, dinv = _tc_first(x, W1, dega, degb)
    s1 = segsum(y1, srcp0, dstp0, srcp1, dstp1, zeros_acc)
    y2 = _tc_mid(s1[0:N], s1[NP:NP + N], y1, dinv,
                 b1.reshape(1, D), g1.reshape(1, D), be1.reshape(1, D), W2)
    s2 = segsum(y2, srcp0, dstp0, srcp1, dstp1, zeros_acc)
    y3 = _tc_mid(s2[0:N], s2[NP:NP + N], y2, dinv,
                 b2.reshape(1, D), g2.reshape(1, D), be2.reshape(1, D), W3)
    s3 = segsum(y3, srcp0, dstp0, srcp1, dstp1, zeros_acc)
    out = _tc_last(s3[0:N], s3[NP:NP + N], y3, dinv, b3.reshape(1, D))
    return out
